# Initial kernel scaffold; baseline (speedup 1.0000x reference)
import jax
import jax.numpy as jnp
from jax.experimental import pallas as pl


def kernel(x, batch_ids):
    raise NotImplementedError("write your pallas kernel here")


# speedup vs baseline: 2.0706x; 2.0706x over previous
import functools

import jax
import jax.numpy as jnp
from jax import lax
from jax.experimental import pallas as pl
from jax.experimental.pallas import tpu as pltpu
from jax.experimental.pallas import tpu_sc as plsc

N_ROWS = 32768
N_COLS = 256
N_SEG = 16
LANES = 16
NC, NS = 2, 16
NW = NC * NS
ROWS_W = N_ROWS // NW
CHUNK = 128
N_CHUNK = ROWS_W // CHUNK
GROUP = 16
N_GROUP = CHUNK // GROUP
CBLK = N_COLS // LANES
SEGC = N_SEG * N_COLS

_mesh = plsc.VectorSubcoreMesh(
    core_axis_name="c", subcore_axis_name="s", num_cores=NC, num_subcores=NS
)


@functools.partial(
    pl.kernel,
    out_type=jax.ShapeDtypeStruct((NW * SEGC,), jnp.float32),
    mesh=_mesh,
    scratch_types=[
        pltpu.VMEM((CHUNK * N_COLS,), jnp.float32),
        pltpu.VMEM((ROWS_W,), jnp.int32),
        pltpu.VMEM((SEGC,), jnp.float32),
    ],
)
def _sc_partial_max(x_hbm, ids_hbm, out_hbm, xbuf, idsbuf, acc):
    cid = lax.axis_index("c")
    sid = lax.axis_index("s")
    wid = sid * NC + cid
    base = wid * ROWS_W

    neg_inf = jnp.full((LANES,), -jnp.inf, jnp.float32)

    def init_body(i, carry):
        acc[pl.ds(pl.multiple_of(i * LANES, LANES), LANES)] = neg_inf
        return carry

    lax.fori_loop(0, SEGC // LANES, init_body, 0)

    pltpu.sync_copy(ids_hbm.at[pl.ds(base, ROWS_W)], idsbuf)

    def chunk_body(ci, carry):
        row0 = base + ci * CHUNK
        pltpu.sync_copy(
            x_hbm.at[pl.ds(pl.multiple_of(row0 * N_COLS, LANES), CHUNK * N_COLS)],
            xbuf,
        )

        def group_body(g, inner):
            lrow = g * GROUP
            id0 = ci * CHUNK + lrow
            idv = idsbuf[pl.ds(pl.multiple_of(id0, LANES), GROUP)]
            lo = idv[0]
            hi = idv[GROUP - 1]
            roff = lrow * N_COLS

            @pl.when(lo == hi)
            def _fast():
                for j in range(CBLK):
                    aoff = pl.multiple_of(lo * N_COLS + j * LANES, LANES)
                    m = acc[pl.ds(aoff, LANES)]
                    for r in range(GROUP):
                        xoff = pl.multiple_of(
                            roff + r * N_COLS + j * LANES, LANES
                        )
                        m = jnp.maximum(m, xbuf[pl.ds(xoff, LANES)])
                    acc[pl.ds(aoff, LANES)] = m

            @pl.when(lo != hi)
            def _slow():
                for r in range(GROUP):
                    seg = idv[r]
                    for j in range(CBLK):
                        aoff = pl.multiple_of(seg * N_COLS + j * LANES, LANES)
                        xoff = pl.multiple_of(
                            roff + r * N_COLS + j * LANES, LANES
                        )
                        a = acc[pl.ds(aoff, LANES)]
                        v = xbuf[pl.ds(xoff, LANES)]
                        acc[pl.ds(aoff, LANES)] = jnp.maximum(a, v)

            return inner

        lax.fori_loop(0, N_GROUP, group_body, 0)
        return carry

    lax.fori_loop(0, N_CHUNK, chunk_body, 0)

    pltpu.sync_copy(acc, out_hbm.at[pl.ds(pl.multiple_of(wid * SEGC, LANES), SEGC)])


def _combine_body(p_ref, o_ref):
    o_ref[:] = jnp.max(p_ref[:], axis=0)


_combine = pl.pallas_call(
    _combine_body,
    out_shape=jax.ShapeDtypeStruct((N_SEG, N_COLS), jnp.float32),
)


@jax.jit
def kernel(x, batch_ids):
    ids = batch_ids.astype(jnp.int32)
    part = _sc_partial_max(x.reshape(-1), ids)
    return _combine(part.reshape(NW, N_SEG, N_COLS))


# speedup vs baseline: 3.6118x; 1.7443x over previous
import functools

import jax
import jax.numpy as jnp
from jax import lax
from jax.experimental import pallas as pl
from jax.experimental.pallas import tpu as pltpu
from jax.experimental.pallas import tpu_sc as plsc

N_ROWS = 32768
N_COLS = 256
N_SEG = 16
LANES = 16
NC, NS = 2, 16
NW = NC * NS
ROWS_W = N_ROWS // NW
CHUNK = 128
N_CHUNK = ROWS_W // CHUNK
GROUP = 16
N_GROUP = CHUNK // GROUP
CBLK = N_COLS // LANES
SEGC = N_SEG * N_COLS

_mesh = plsc.VectorSubcoreMesh(
    core_axis_name="c", subcore_axis_name="s", num_cores=NC, num_subcores=NS
)


def _xy(r, j):
    return r, j * LANES


@functools.partial(
    pl.kernel,
    out_type=jax.ShapeDtypeStruct((NW * SEGC,), jnp.float32),
    mesh=_mesh,
    compiler_params=pltpu.CompilerParams(use_tc_tiling_on_sc=True),
    scratch_types=[
        pltpu.VMEM((CHUNK, N_COLS), jnp.float32),
        pltpu.VMEM((CHUNK, N_COLS), jnp.float32),
        pltpu.VMEM((ROWS_W,), jnp.int32),
        pltpu.VMEM((SEGC,), jnp.float32),
        pltpu.SemaphoreType.DMA,
        pltpu.SemaphoreType.DMA,
    ],
)
def _sc_partial_max(x_hbm, ids_hbm, out_hbm, xbufa, xbufb, idsbuf, acc, sema, semb):
    cid = lax.axis_index("c")
    sid = lax.axis_index("s")
    wid = sid * NC + cid
    base = wid * ROWS_W

    neg_inf = jnp.full((LANES,), -jnp.inf, jnp.float32)

    def init_body(i, carry):
        acc[pl.ds(pl.multiple_of(i * LANES, LANES), LANES)] = neg_inf
        return carry

    lax.fori_loop(0, SEGC // LANES, init_body, 0)

    pltpu.sync_copy(ids_hbm.at[pl.ds(base, ROWS_W)], idsbuf)

    def _chunk_src(ci):
        row0 = base + ci * CHUNK
        return x_hbm.at[pl.ds(pl.multiple_of(row0, CHUNK), CHUNK), :]

    def _start(ci, buf, sem):
        pltpu.make_async_copy(_chunk_src(ci), buf, sem).start()

    def _wait(buf, sem):
        pltpu.make_async_copy(_chunk_src(0), buf, sem).wait()

    def _process(ci, xbuf):
        def group_body(g, inner):
            lrow = pl.multiple_of(g * GROUP, GROUP)
            id0 = ci * CHUNK + lrow
            idv = idsbuf[pl.ds(pl.multiple_of(id0, LANES), GROUP)]
            lo = idv[0]
            hi = idv[GROUP - 1]

            @pl.when(lo == hi)
            def _fast():
                for j in range(CBLK):
                    aoff = pl.multiple_of(lo * N_COLS + j * LANES, LANES)
                    m = acc[pl.ds(aoff, LANES)]
                    for r in range(GROUP):
                        dr, col = _xy(r, j)
                        m = jnp.maximum(m, xbuf[lrow + dr, pl.ds(col, LANES)])
                    acc[pl.ds(aoff, LANES)] = m

            @pl.when(lo != hi)
            def _slow():
                for r in range(GROUP):
                    seg = idv[r]
                    for j in range(CBLK):
                        aoff = pl.multiple_of(seg * N_COLS + j * LANES, LANES)
                        dr, col = _xy(r, j)
                        a = acc[pl.ds(aoff, LANES)]
                        v = xbuf[lrow + dr, pl.ds(col, LANES)]
                        acc[pl.ds(aoff, LANES)] = jnp.maximum(a, v)

            return inner

        lax.fori_loop(0, N_GROUP, group_body, 0)

    _start(0, xbufa, sema)

    def outer_body(i, carry):
        ci = pl.multiple_of(i * 2, 2)
        _start(ci + 1, xbufb, semb)
        _wait(xbufa, sema)
        _process(ci, xbufa)

        @pl.when(ci + 2 < N_CHUNK)
        def _prefetch_a():
            _start(ci + 2, xbufa, sema)

        _wait(xbufb, semb)
        _process(ci + 1, xbufb)
        return carry

    lax.fori_loop(0, N_CHUNK // 2, outer_body, 0)

    pltpu.sync_copy(acc, out_hbm.at[pl.ds(pl.multiple_of(wid * SEGC, LANES), SEGC)])


def _combine_body(p_ref, o_ref):
    o_ref[:] = jnp.max(p_ref[:], axis=0)


_combine = pl.pallas_call(
    _combine_body,
    out_shape=jax.ShapeDtypeStruct((N_SEG, N_COLS), jnp.float32),
)


@jax.jit
def kernel(x, batch_ids):
    ids = batch_ids.astype(jnp.int32)
    part = _sc_partial_max(x, ids)
    return _combine(part.reshape(NW, N_SEG, N_COLS))


# speedup vs baseline: 3.8090x; 1.0546x over previous
import functools

import jax
import jax.numpy as jnp
from jax import lax
from jax.experimental import pallas as pl
from jax.experimental.pallas import tpu as pltpu
from jax.experimental.pallas import tpu_sc as plsc

N_ROWS = 32768
N_COLS = 256
N_SEG = 16
LANES = 16
NC, NS = 2, 16
NW = NC * NS
ROWS_W = N_ROWS // NW
CHUNK = 128
N_CHUNK = ROWS_W // CHUNK
GROUP = 16
N_GROUP = CHUNK // GROUP
CBLK = N_COLS // LANES
SEGC = N_SEG * N_COLS

_mesh = plsc.VectorSubcoreMesh(
    core_axis_name="c", subcore_axis_name="s", num_cores=NC, num_subcores=NS
)


def _xy(r, j):
    return r, j * LANES


@functools.partial(
    pl.kernel,
    out_type=jax.ShapeDtypeStruct((NW, N_SEG, N_COLS), jnp.float32),
    mesh=_mesh,
    compiler_params=pltpu.CompilerParams(use_tc_tiling_on_sc=True),
    scratch_types=[
        pltpu.VMEM((CHUNK, N_COLS), jnp.float32),
        pltpu.VMEM((CHUNK, N_COLS), jnp.float32),
        pltpu.VMEM((ROWS_W,), jnp.int32),
        pltpu.VMEM((N_SEG, N_COLS), jnp.float32),
        pltpu.SemaphoreType.DMA,
        pltpu.SemaphoreType.DMA,
    ],
)
def _sc_partial_max(x_hbm, ids_hbm, out_hbm, xbufa, xbufb, idsbuf, acc, sema, semb):
    cid = lax.axis_index("c")
    sid = lax.axis_index("s")
    wid = sid * NC + cid
    base = wid * ROWS_W

    neg_inf = jnp.full((LANES,), -jnp.inf, jnp.float32)

    def init_body(j, carry):
        col = pl.multiple_of(j * LANES, LANES)
        for s in range(N_SEG):
            acc[s, pl.ds(col, LANES)] = neg_inf
        return carry

    lax.fori_loop(0, CBLK, init_body, 0)

    pltpu.sync_copy(ids_hbm.at[pl.ds(base, ROWS_W)], idsbuf)

    def _chunk_src(ci):
        row0 = base + ci * CHUNK
        return x_hbm.at[pl.ds(pl.multiple_of(row0, CHUNK), CHUNK), :]

    def _start(ci, buf, sem):
        pltpu.make_async_copy(_chunk_src(ci), buf, sem).start()

    def _wait(buf, sem):
        pltpu.make_async_copy(_chunk_src(0), buf, sem).wait()

    def _process(ci, xbuf):
        def group_body(g, inner):
            lrow = pl.multiple_of(g * GROUP, GROUP)
            id0 = ci * CHUNK + lrow
            idv = idsbuf[pl.ds(pl.multiple_of(id0, LANES), GROUP)]
            lo = idv[0]
            hi = idv[GROUP - 1]

            @pl.when(lo == hi)
            def _fast():
                for j in range(CBLK):
                    col = j * LANES
                    m = acc[lo, pl.ds(col, LANES)]
                    for r in range(GROUP):
                        dr, xcol = _xy(r, j)
                        m = jnp.maximum(m, xbuf[lrow + dr, pl.ds(xcol, LANES)])
                    acc[lo, pl.ds(col, LANES)] = m

            @pl.when(lo != hi)
            def _slow():
                for r in range(GROUP):
                    seg = idv[r]
                    for j in range(CBLK):
                        col = j * LANES
                        dr, xcol = _xy(r, j)
                        a = acc[seg, pl.ds(col, LANES)]
                        v = xbuf[lrow + dr, pl.ds(xcol, LANES)]
                        acc[seg, pl.ds(col, LANES)] = jnp.maximum(a, v)

            return inner

        lax.fori_loop(0, N_GROUP, group_body, 0)

    _start(0, xbufa, sema)

    def outer_body(i, carry):
        ci = pl.multiple_of(i * 2, 2)
        _start(ci + 1, xbufb, semb)
        _wait(xbufa, sema)
        _process(ci, xbufa)

        @pl.when(ci + 2 < N_CHUNK)
        def _prefetch_a():
            _start(ci + 2, xbufa, sema)

        _wait(xbufb, semb)
        _process(ci + 1, xbufb)
        return carry

    lax.fori_loop(0, N_CHUNK // 2, outer_body, 0)

    pltpu.sync_copy(acc, out_hbm.at[wid])


def _combine_body(p_ref, o_ref):
    o_ref[:] = jnp.max(p_ref[:], axis=0)


_combine = pl.pallas_call(
    _combine_body,
    out_shape=jax.ShapeDtypeStruct((N_SEG, N_COLS), jnp.float32),
)


@jax.jit
def kernel(x, batch_ids):
    ids = batch_ids.astype(jnp.int32)
    part = _sc_partial_max(x, ids)
    return _combine(part)
